# baseline (speedup 1.0000x reference)
import functools

import jax
import jax.numpy as jnp
from jax.experimental import pallas as pl

NUM_EXPERTS = 64
TOP_K = 8
BALANCE_FACTOR = 1e-4


def _router_body(x1_ref, x2_ref, w_ref, b_ref, dm_ref, idx_ref, load_ref,
                 loss_ref, *, num_tiles, total_tokens):
    i = pl.program_id(0)
    w = w_ref[...]
    dn = (((1,), (1,)), ((), ()))
    lt1 = jax.lax.dot_general(w, x1_ref[...], dimension_numbers=dn,
                              preferred_element_type=jnp.float32)
    lt2 = jax.lax.dot_general(w, x2_ref[...], dimension_numbers=dn,
                              preferred_element_type=jnp.float32)
    lt = jnp.concatenate([lt1, lt2], axis=1) + b_ref[...]

    m = jnp.max(lt, axis=0, keepdims=True)
    e = jnp.exp(lt - m)
    probs = e / jnp.sum(e, axis=0, keepdims=True)

    rows_f = jax.lax.broadcasted_iota(jnp.int32, probs.shape, 0).astype(
        jnp.float32)
    work = probs
    idx_rows = []
    for _ in range(TOP_K):
        mk = jnp.max(work, axis=0, keepdims=True)
        amax = jnp.min(jnp.where(work == mk, rows_f, float(NUM_EXPERTS)),
                       axis=0, keepdims=True)
        idx_rows.append(amax)
        work = jnp.where(rows_f == amax, -1.0, work)

    idx_ref[...] = jnp.concatenate(idx_rows, axis=0).astype(jnp.int32)
    psel = jnp.where(work < 0.0, probs, 0.0)
    dmt = psel / jnp.sum(psel, axis=0, keepdims=True)
    r64 = jax.lax.broadcasted_iota(jnp.int32, (NUM_EXPERTS, NUM_EXPERTS), 0)
    c64 = jax.lax.broadcasted_iota(jnp.int32, (NUM_EXPERTS, NUM_EXPERTS), 1)
    eye = (r64 == c64).astype(jnp.float32)
    dm_ref[...] = jax.lax.dot_general(
        dmt, eye, dimension_numbers=(((0,), (0,)), ((), ())),
        precision=jax.lax.Precision.HIGHEST,
        preferred_element_type=jnp.float32)

    part = jnp.sum(probs, axis=1, keepdims=True)

    @pl.when(i == 0)
    def _init():
        load_ref[...] = part

    @pl.when(i > 0)
    def _acc():
        load_ref[...] = load_ref[...] + part

    @pl.when(i == num_tiles - 1)
    def _finish():
        load = load_ref[...] / total_tokens
        target = 1.0 / NUM_EXPERTS
        kl = target * (jnp.log(target) - jnp.log(load))
        loss_ref[...] = jnp.sum(kl, axis=0, keepdims=True) * (
            BALANCE_FACTOR / NUM_EXPERTS)


def kernel(hidden_states, W, expert_bias):
    Bb, Ss, Dd = hidden_states.shape
    T = Bb * Ss
    TT = 1024
    TH = TT // 2
    num_tiles = T // TT

    x = hidden_states.reshape(T, Dd)
    bias = expert_bias.reshape(NUM_EXPERTS, 1)

    body = functools.partial(_router_body, num_tiles=num_tiles,
                             total_tokens=float(T))

    dmt, idxt, _, loss = pl.pallas_call(
        body,
        grid=(num_tiles,),
        in_specs=[
            pl.BlockSpec((TH, Dd), lambda i: (2 * i, 0)),
            pl.BlockSpec((TH, Dd), lambda i: (2 * i + 1, 0)),
            pl.BlockSpec((NUM_EXPERTS, Dd), lambda i: (0, 0)),
            pl.BlockSpec((NUM_EXPERTS, 1), lambda i: (0, 0)),
        ],
        out_specs=[
            pl.BlockSpec((TT, NUM_EXPERTS), lambda i: (i, 0)),
            pl.BlockSpec((TOP_K, TT), lambda i: (0, i)),
            pl.BlockSpec((NUM_EXPERTS, 1), lambda i: (0, 0)),
            pl.BlockSpec((1, 1), lambda i: (0, 0)),
        ],
        out_shape=[
            jax.ShapeDtypeStruct((T, NUM_EXPERTS), jnp.float32),
            jax.ShapeDtypeStruct((TOP_K, T), jnp.int32),
            jax.ShapeDtypeStruct((NUM_EXPERTS, 1), jnp.float32),
            jax.ShapeDtypeStruct((1, 1), jnp.float32),
        ],
    )(x, x, W, bias)

    dispatch_mask = dmt.reshape(Bb, Ss, NUM_EXPERTS)
    top_k_indices = idxt.T.reshape(Bb, Ss, TOP_K)
    balance_loss = loss.reshape(())
    return dispatch_mask, balance_loss, top_k_indices


# speedup vs baseline: 1.1272x; 1.1272x over previous
import functools

import jax
import jax.numpy as jnp
from jax.experimental import pallas as pl

NUM_EXPERTS = 64
TOP_K = 8
BALANCE_FACTOR = 1e-4


def _router_body(x1_ref, x2_ref, w_ref, b_ref, dm_ref, idx_ref, load_ref,
                 loss_ref, *, num_tiles, total_tokens):
    i = pl.program_id(0)
    w = w_ref[...]
    dn = (((1,), (1,)), ((), ()))
    lt1 = jax.lax.dot_general(w, x1_ref[...], dimension_numbers=dn,
                              preferred_element_type=jnp.float32)
    lt2 = jax.lax.dot_general(w, x2_ref[...], dimension_numbers=dn,
                              preferred_element_type=jnp.float32)
    lt = jnp.concatenate([lt1, lt2], axis=1) + b_ref[...]

    m = jnp.max(lt, axis=0, keepdims=True)
    e = jnp.exp(lt - m)
    probs = e / jnp.sum(e, axis=0, keepdims=True)

    rows_f = jax.lax.broadcasted_iota(jnp.int32, probs.shape, 0).astype(
        jnp.float32)
    work = probs
    idx_rows = []
    for _ in range(TOP_K):
        mk = jnp.max(work, axis=0, keepdims=True)
        amax = jnp.min(jnp.where(work == mk, rows_f, float(NUM_EXPERTS)),
                       axis=0, keepdims=True)
        idx_rows.append(amax)
        work = jnp.where(rows_f == amax, -1.0, work)

    idx_ref[...] = jnp.concatenate(idx_rows, axis=0).astype(jnp.int32)
    psel = jnp.where(work < 0.0, probs, 0.0)
    dm_ref[...] = psel / jnp.sum(psel, axis=0, keepdims=True)

    part = jnp.sum(probs, axis=1, keepdims=True)

    @pl.when(i == 0)
    def _init():
        load_ref[...] = part

    @pl.when(i > 0)
    def _acc():
        load_ref[...] = load_ref[...] + part

    @pl.when(i == num_tiles - 1)
    def _finish():
        load = load_ref[...] / total_tokens
        target = 1.0 / NUM_EXPERTS
        kl = target * (jnp.log(target) - jnp.log(load))
        loss_ref[...] = jnp.sum(kl, axis=0, keepdims=True) * (
            BALANCE_FACTOR / NUM_EXPERTS)


def kernel(hidden_states, W, expert_bias):
    Bb, Ss, Dd = hidden_states.shape
    T = Bb * Ss
    TT = 2048
    TH = TT // 2
    num_tiles = T // TT

    x = hidden_states.reshape(T, Dd)
    bias = expert_bias.reshape(NUM_EXPERTS, 1)

    body = functools.partial(_router_body, num_tiles=num_tiles,
                             total_tokens=float(T))

    dmt, idxt, _, loss = pl.pallas_call(
        body,
        grid=(num_tiles,),
        in_specs=[
            pl.BlockSpec((TH, Dd), lambda i: (2 * i, 0)),
            pl.BlockSpec((TH, Dd), lambda i: (2 * i + 1, 0)),
            pl.BlockSpec((NUM_EXPERTS, Dd), lambda i: (0, 0)),
            pl.BlockSpec((NUM_EXPERTS, 1), lambda i: (0, 0)),
        ],
        out_specs=[
            pl.BlockSpec((NUM_EXPERTS, TT), lambda i: (0, i)),
            pl.BlockSpec((TOP_K, TT), lambda i: (0, i)),
            pl.BlockSpec((NUM_EXPERTS, 1), lambda i: (0, 0)),
            pl.BlockSpec((1, 1), lambda i: (0, 0)),
        ],
        out_shape=[
            jax.ShapeDtypeStruct((NUM_EXPERTS, T), jnp.float32),
            jax.ShapeDtypeStruct((TOP_K, T), jnp.int32),
            jax.ShapeDtypeStruct((NUM_EXPERTS, 1), jnp.float32),
            jax.ShapeDtypeStruct((1, 1), jnp.float32),
        ],
    )(x, x, W, bias)

    dispatch_mask = dmt.T.reshape(Bb, Ss, NUM_EXPERTS)
    top_k_indices = idxt.T.reshape(Bb, Ss, TOP_K)
    balance_loss = loss.reshape(())
    return dispatch_mask, balance_loss, top_k_indices


# speedup vs baseline: 1.1430x; 1.0140x over previous
import functools

import jax
import jax.numpy as jnp
from jax.experimental import pallas as pl

NUM_EXPERTS = 64
TOP_K = 8
BALANCE_FACTOR = 1e-4


def _router_body(x1_ref, x2_ref, w_ref, b_ref, dm_ref, idx_ref, load_ref,
                 loss_ref, *, num_tiles, total_tokens):
    i = pl.program_id(0)
    w = w_ref[...]
    dn = (((1,), (1,)), ((), ()))
    lt1 = jax.lax.dot_general(w, x1_ref[...], dimension_numbers=dn,
                              preferred_element_type=jnp.float32)
    lt2 = jax.lax.dot_general(w, x2_ref[...], dimension_numbers=dn,
                              preferred_element_type=jnp.float32)
    lt = jnp.concatenate([lt1, lt2], axis=1) + b_ref[...]

    m = jnp.max(lt, axis=0, keepdims=True)
    e = jnp.exp(lt - m)
    probs = e / jnp.sum(e, axis=0, keepdims=True)

    rows_f = jax.lax.broadcasted_iota(jnp.int32, probs.shape, 0).astype(
        jnp.float32)
    work = probs
    idx_rows = []
    for _ in range(TOP_K):
        mk = jnp.max(work, axis=0, keepdims=True)
        amax = jnp.min(jnp.where(work == mk, rows_f, float(NUM_EXPERTS)),
                       axis=0, keepdims=True)
        idx_rows.append(amax)
        work = jnp.where(rows_f == amax, -1.0, work)

    idx_ref[...] = jnp.concatenate(idx_rows, axis=0).astype(jnp.int32)
    psel = jnp.where(work < 0.0, probs, 0.0)
    dm_ref[...] = psel / jnp.sum(psel, axis=0, keepdims=True)

    part = jnp.sum(probs, axis=1, keepdims=True)

    @pl.when(i == 0)
    def _init():
        load_ref[...] = part

    @pl.when(i > 0)
    def _acc():
        load_ref[...] = load_ref[...] + part

    @pl.when(i == num_tiles - 1)
    def _finish():
        load = load_ref[...] / total_tokens
        target = 1.0 / NUM_EXPERTS
        kl = target * (jnp.log(target) - jnp.log(load))
        loss_ref[...] = jnp.sum(kl, axis=0, keepdims=True) * (
            BALANCE_FACTOR / NUM_EXPERTS)


def kernel(hidden_states, W, expert_bias):
    Bb, Ss, Dd = hidden_states.shape
    T = Bb * Ss
    TT = 1024
    TH = TT // 2
    num_tiles = T // TT

    x = hidden_states.reshape(T, Dd)
    bias = expert_bias.reshape(NUM_EXPERTS, 1)

    body = functools.partial(_router_body, num_tiles=num_tiles,
                             total_tokens=float(T))

    dmt, idxt, _, loss = pl.pallas_call(
        body,
        grid=(num_tiles,),
        in_specs=[
            pl.BlockSpec((TH, Dd), lambda i: (2 * i, 0)),
            pl.BlockSpec((TH, Dd), lambda i: (2 * i + 1, 0)),
            pl.BlockSpec((NUM_EXPERTS, Dd), lambda i: (0, 0)),
            pl.BlockSpec((NUM_EXPERTS, 1), lambda i: (0, 0)),
        ],
        out_specs=[
            pl.BlockSpec((NUM_EXPERTS, TT), lambda i: (0, i)),
            pl.BlockSpec((TOP_K, TT), lambda i: (0, i)),
            pl.BlockSpec((NUM_EXPERTS, 1), lambda i: (0, 0)),
            pl.BlockSpec((1, 1), lambda i: (0, 0)),
        ],
        out_shape=[
            jax.ShapeDtypeStruct((NUM_EXPERTS, T), jnp.float32),
            jax.ShapeDtypeStruct((TOP_K, T), jnp.int32),
            jax.ShapeDtypeStruct((NUM_EXPERTS, 1), jnp.float32),
            jax.ShapeDtypeStruct((1, 1), jnp.float32),
        ],
    )(x, x, W, bias)

    dispatch_mask = dmt.T.reshape(Bb, Ss, NUM_EXPERTS)
    top_k_indices = idxt.T.reshape(Bb, Ss, TOP_K)
    balance_loss = loss.reshape(())
    return dispatch_mask, balance_loss, top_k_indices
